# baseline (speedup 1.0000x reference)
import functools

import jax
import jax.numpy as jnp
from jax import lax
from jax.experimental import pallas as pl
from jax.experimental.pallas import tpu as pltpu
from jax.experimental.pallas import tpu_sc as plsc

N = 10000
E = 320000
D = 128
NP = 10240
NC = 2
NS = 16
NW = NC * NS
L = 16

CHUNK = 128
NCH = E // CHUNK
CPW = NCH // NW
REM = NCH - CPW * NW
CMAX = CPW + 1
EPW_MAX = CMAX * CHUNK
E_PAD = (NCH + 1) * CHUNK
SLICE = NP // NS

_mesh = plsc.VectorSubcoreMesh(
    core_axis_name="c", subcore_axis_name="s", num_cores=NC, num_subcores=NS)


def _wid_base_count(c, s):
    wid = s * NC + c
    cbase = wid * CPW + jnp.minimum(wid, REM)
    ccount = CPW + (wid < REM).astype(jnp.int32)
    return wid, cbase, ccount


@functools.partial(
    pl.kernel,
    out_type=jax.ShapeDtypeStruct((NC, 1, NP), jnp.float32),
    mesh=_mesh,
    compiler_params=pltpu.CompilerParams(needs_layout_passes=False),
    scratch_types=[
        pltpu.VMEM((EPW_MAX,), jnp.int32),
        pltpu.VMEM((EPW_MAX,), jnp.float32),
        pltpu.VMEM((NP,), jnp.float32),
        pltpu.VMEM((NS, SLICE), jnp.float32),
        pltpu.VMEM((SLICE,), jnp.float32),
        pltpu.VMEM_SHARED((NS, 1, NP), jnp.float32),
    ],
)
def _deg_kernel(dst_hbm, ew_hbm, degp_hbm, dstl, ewl, hist, redbuf, outbuf,
                shared):
    c = lax.axis_index("c")
    s = lax.axis_index("s")
    _, cbase, ccount = _wid_base_count(c, s)

    zero16 = jnp.zeros((L,), jnp.float32)

    def zero_body(i, _):
        hist[pl.ds(i * L, L)] = zero16
        return 0
    lax.fori_loop(0, NP // L, zero_body, 0)

    ebase = cbase * CHUNK
    pltpu.sync_copy(dst_hbm.at[pl.ds(ebase, EPW_MAX)], dstl)
    pltpu.sync_copy(ew_hbm.at[pl.ds(ebase, EPW_MAX)], ewl)

    def hist_body(g, _):
        dv = dstl[pl.ds(g * L, L)]
        wv = ewl[pl.ds(g * L, L)]
        plsc.addupdate_scatter(hist, [dv], wv)
        return 0
    lax.fori_loop(0, ccount * (CHUNK // L), hist_body, 0)

    pltpu.sync_copy(hist, shared.at[s, 0])
    plsc.subcore_barrier()

    pltpu.sync_copy(shared.at[:, 0, pl.ds(s * SLICE, SLICE)], redbuf)

    for i in range(SLICE // L):
        sl = pl.ds(i * L, L)
        acc = redbuf[0, sl]
        for r in range(1, NS):
            acc = acc + redbuf[r, sl]
        outbuf[sl] = acc
    pltpu.sync_copy(outbuf, degp_hbm.at[c, 0, pl.ds(s * SLICE, SLICE)])


PACKW = 3 * CHUNK


@functools.partial(
    pl.kernel,
    out_type=jax.ShapeDtypeStruct((E_PAD // CHUNK, 1, PACKW), jnp.int32),
    mesh=_mesh,
    compiler_params=pltpu.CompilerParams(needs_layout_passes=False),
    scratch_types=[
        pltpu.VMEM((EPW_MAX,), jnp.int32),
        pltpu.VMEM((EPW_MAX,), jnp.int32),
        pltpu.VMEM((EPW_MAX,), jnp.float32),
        pltpu.VMEM((CMAX, 1, PACKW), jnp.int32),
        pltpu.VMEM((NP,), jnp.float32),
    ],
)
def _norm_kernel(src_hbm, dst_hbm, ew_hbm, dis_hbm, edata_hbm, srcl, dstl,
                 ewl, epack, disv):
    c = lax.axis_index("c")
    s = lax.axis_index("s")
    wid, cbase, ccount = _wid_base_count(c, s)

    pltpu.sync_copy(dis_hbm, disv)
    ebase = cbase * CHUNK
    pltpu.sync_copy(src_hbm.at[pl.ds(ebase, EPW_MAX)], srcl)
    pltpu.sync_copy(dst_hbm.at[pl.ds(ebase, EPW_MAX)], dstl)
    pltpu.sync_copy(ew_hbm.at[pl.ds(ebase, EPW_MAX)], ewl)

    def body(j, _):
        for g in range(CHUNK // L):
            sl = pl.ds(g * L, L)
            s16 = srcl[pl.ds(j * CHUNK + g * L, L)]
            d16 = dstl[pl.ds(j * CHUNK + g * L, L)]
            w16 = ewl[pl.ds(j * CHUNK + g * L, L)]
            nrm = plsc.load_gather(disv, [s16]) * w16 * \
                plsc.load_gather(disv, [d16])
            epack[j, 0, sl] = s16
            epack[j, 0, pl.ds(CHUNK + g * L, L)] = d16
            epack[j, 0, pl.ds(2 * CHUNK + g * L, L)] = plsc.bitcast(
                nrm, jnp.int32)
        return 0
    lax.fori_loop(0, ccount, body, 0)

    pltpu.sync_copy(epack.at[pl.ds(0, CPW)], edata_hbm.at[pl.ds(cbase, CPW)])

    @pl.when(wid < REM)
    def _():
        pltpu.sync_copy(epack.at[pl.ds(CPW, 1)],
                        edata_hbm.at[pl.ds(cbase + CPW, 1)])


ROWS_A = 632
ROWS_LAST = N - 15 * ROWS_A


def _acc_slices(s):
    base = s * ROWS_A
    return base


@functools.partial(
    pl.kernel,
    out_type=jax.ShapeDtypeStruct((NC, N, D), jnp.float32),
    mesh=_mesh,
    compiler_params=pltpu.CompilerParams(needs_layout_passes=False),
    scratch_types=[
        pltpu.VMEM((1, PACKW), jnp.int32),
        pltpu.VMEM((1, PACKW), jnp.int32),
        pltpu.VMEM((1, PACKW), jnp.int32),
        pltpu.VMEM((1, PACKW), jnp.int32),
        pltpu.VMEM((CHUNK, D), jnp.float32),
        pltpu.VMEM((CHUNK, D), jnp.float32),
        pltpu.VMEM((CHUNK, D), jnp.float32),
        pltpu.VMEM_SHARED((N, D), jnp.float32),
        pltpu.SemaphoreType.DMA,
        pltpu.SemaphoreType.DMA,
        pltpu.SemaphoreType.DMA,
        pltpu.SemaphoreType.DMA,
        pltpu.SemaphoreType.DMA,
        pltpu.SemaphoreType.DMA,
        pltpu.SemaphoreType.DMA,
        pltpu.SemaphoreType.DMA,
        pltpu.SemaphoreType.DMA,
        pltpu.SemaphoreType.DMA,
    ],
)
def _agg_kernel(h_hbm, edata_hbm, accp_hbm,
                ebA, ebB, ebC, ebD, msgA, msgB, msgC, acc_sh,
                csA, csB, csC, csD, gsA, gsB, gsC, ssA, ssB, ssC):
    c = lax.axis_index("c")
    s = lax.axis_index("s")
    _, cbase, ccount = _wid_base_count(c, s)

    zero16 = jnp.zeros((L,), jnp.float32)

    def zbody(i, _):
        for q in range(D // L):
            msgA[i, pl.ds(q * L, L)] = zero16
        return 0
    lax.fori_loop(0, CHUNK, zbody, 0)
    base = s * ROWS_A
    for k in range(4):
        pltpu.sync_copy(msgA, acc_sh.at[pl.ds(base + k * CHUNK, CHUNK)])

    @pl.when(s < 15)
    def _():
        pltpu.sync_copy(msgA.at[pl.ds(0, ROWS_A - 4 * CHUNK)],
                        acc_sh.at[pl.ds(base + 4 * CHUNK,
                                        ROWS_A - 4 * CHUNK)])

    @pl.when(s == 15)
    def _():
        pltpu.sync_copy(msgA.at[pl.ds(0, ROWS_LAST - 4 * CHUNK)],
                        acc_sh.at[pl.ds(base + 4 * CHUNK,
                                        ROWS_LAST - 4 * CHUNK)])
    plsc.subcore_barrier()

    pltpu.async_copy(edata_hbm.at[cbase], ebA, csA)
    pltpu.async_copy(edata_hbm.at[cbase + 1], ebB, csB)
    pltpu.make_async_copy(edata_hbm.at[cbase], ebA, csA).wait()
    pltpu.async_copy(h_hbm.at[ebA.at[0, pl.ds(0, CHUNK)]], msgA, gsA)

    ebufs = ((ebA, csA), (ebB, csB), (ebC, csC), (ebD, csD))
    mbufs = ((msgA, gsA, ssA), (msgB, gsB, ssB), (msgC, gsC, ssC))
    zz = jnp.zeros((L,), jnp.int32)

    def slot_body(ii, _):
        for t in range(12):
            i = ii * 12 + t
            eb_i, cs_i = ebufs[t % 4]
            msg_i, gs_i, ss_i = mbufs[t % 3]
            eb_n, cs_n = ebufs[(t + 1) % 4]
            msg_n, gs_n, ss_n = mbufs[(t + 1) % 3]
            eb_c, cs_c = ebufs[(t + 2) % 4]

            @pl.when(i < ccount)
            def _():
                @pl.when(i + 1 < ccount)
                def _():
                    pltpu.make_async_copy(
                        edata_hbm.at[cbase + i + 1], eb_n, cs_n).wait()

                    @pl.when(i >= 2)
                    def _():
                        pltpu.make_async_copy(
                            msg_n,
                            acc_sh.at[eb_c.at[0, pl.ds(CHUNK, CHUNK)]],
                            ss_n).wait()
                    pltpu.async_copy(
                        h_hbm.at[eb_n.at[0, pl.ds(0, CHUNK)]], msg_n, gs_n)

                @pl.when(i + 2 < ccount)
                def _():
                    pltpu.async_copy(
                        edata_hbm.at[cbase + i + 2], eb_c, cs_c)

                pltpu.make_async_copy(
                    h_hbm.at[eb_i.at[0, pl.ds(0, CHUNK)]], msg_i,
                    gs_i).wait()

                @plsc.parallel_loop(0, CHUNK, unroll=8)
                def _(k):
                    ns = plsc.bitcast(
                        plsc.load_gather(
                            eb_i, [zz, jnp.full((L,), 2 * CHUNK + k,
                                                jnp.int32)]),
                        jnp.float32)
                    for q in range(D // L):
                        sl = pl.ds(q * L, L)
                        msg_i[k, sl] = msg_i[k, sl] * ns

                pltpu.async_copy(msg_i,
                                 acc_sh.at[eb_i.at[0, pl.ds(CHUNK, CHUNK)]],
                                 ss_i, add=True)
        return 0
    lax.fori_loop(0, (ccount + 11) // 12, slot_body, 0)

    for t in range(3):
        pltpu.make_async_copy(
            mbufs[t][0], acc_sh.at[ebA.at[0, pl.ds(CHUNK, CHUNK)]],
            mbufs[t][2]).wait()

    plsc.subcore_barrier()

    @pl.when(s < 15)
    def _():
        pltpu.sync_copy(acc_sh.at[pl.ds(base, ROWS_A)],
                        accp_hbm.at[c, pl.ds(base, ROWS_A)])

    @pl.when(s == 15)
    def _():
        pltpu.sync_copy(acc_sh.at[pl.ds(base, ROWS_LAST)],
                        accp_hbm.at[c, pl.ds(base, ROWS_LAST)])


def _dis_body(degp_ref, dis_ref, idg_ref):
    deg = degp_ref[0] + degp_ref[1] + 1.0
    dis_ref[...] = lax.rsqrt(deg)
    idg_ref[...] = 1.0 / deg


def _dis_kernel(degp):
    return pl.pallas_call(
        _dis_body,
        out_shape=[jax.ShapeDtypeStruct((NP // 128, 128), jnp.float32),
                   jax.ShapeDtypeStruct((NP // 128, 128), jnp.float32)],
    )(degp.reshape(NC, NP // 128, 128))


_BLK = 2000


def _mm_body(x_ref, w_ref, o_ref):
    o_ref[...] = jnp.dot(x_ref[...], w_ref[...],
                         preferred_element_type=jnp.float32)


def _mm(x, w):
    n = x.shape[0]
    return pl.pallas_call(
        _mm_body,
        grid=(n // _BLK,),
        in_specs=[pl.BlockSpec((_BLK, D), lambda i: (i, 0)),
                  pl.BlockSpec((D, D), lambda i: (0, 0))],
        out_specs=pl.BlockSpec((_BLK, D), lambda i: (i, 0)),
        out_shape=jax.ShapeDtypeStruct((n, D), jnp.float32),
    )(x, w)


def _comb_body(accp_ref, h_ref, idg_ref, b_ref, w_ref, o_ref):
    z = (accp_ref[0] + accp_ref[1] + idg_ref[...] * h_ref[...] + b_ref[...])
    z = jnp.maximum(z, 0.0)
    o_ref[...] = jnp.dot(z, w_ref[...], preferred_element_type=jnp.float32)


def _combine_mm(accp, h, idgcol, b, w):
    return pl.pallas_call(
        _comb_body,
        grid=(N // _BLK,),
        in_specs=[pl.BlockSpec((NC, _BLK, D), lambda i: (0, i, 0)),
                  pl.BlockSpec((_BLK, D), lambda i: (i, 0)),
                  pl.BlockSpec((_BLK, 1), lambda i: (i, 0)),
                  pl.BlockSpec((1, D), lambda i: (0, 0)),
                  pl.BlockSpec((D, D), lambda i: (0, 0))],
        out_specs=pl.BlockSpec((_BLK, D), lambda i: (i, 0)),
        out_shape=jax.ShapeDtypeStruct((N, D), jnp.float32),
    )(accp, h, idgcol, b.reshape(1, D), w)


def _head_body(accp_ref, h_ref, idg_ref, b_ref, wm1_ref, bm1_ref, wm2_ref,
               bm2_ref, o_ref):
    z = (accp_ref[0] + accp_ref[1] + idg_ref[...] * h_ref[...] + b_ref[...])
    z = jnp.maximum(z, 0.0)
    m = jnp.dot(z, wm1_ref[...], preferred_element_type=jnp.float32)
    m = jnp.maximum(m + bm1_ref[...], 0.0)
    o = jnp.dot(m, wm2_ref[...], preferred_element_type=jnp.float32)
    o_ref[...] = jax.nn.sigmoid(o + bm2_ref[...])


def _head(accp, h, idgcol, b2, wm1, bm1, wm2p, bm2p):
    return pl.pallas_call(
        _head_body,
        grid=(N // _BLK,),
        in_specs=[pl.BlockSpec((NC, _BLK, D), lambda i: (0, i, 0)),
                  pl.BlockSpec((_BLK, D), lambda i: (i, 0)),
                  pl.BlockSpec((_BLK, 1), lambda i: (i, 0)),
                  pl.BlockSpec((1, D), lambda i: (0, 0)),
                  pl.BlockSpec((D, D), lambda i: (0, 0)),
                  pl.BlockSpec((1, D), lambda i: (0, 0)),
                  pl.BlockSpec((D, D), lambda i: (0, 0)),
                  pl.BlockSpec((1, D), lambda i: (0, 0))],
        out_specs=pl.BlockSpec((_BLK, D), lambda i: (i, 0)),
        out_shape=jax.ShapeDtypeStruct((N, D), jnp.float32),
    )(accp, h, idgcol, b2.reshape(1, D), wm1, bm1.reshape(1, D), wm2p,
      bm2p.reshape(1, D))


def kernel(x, edge_index, edge_attr, W1, b1, W2, b2, Wm1, bm1, Wm2, bm2):
    src = edge_index[0].astype(jnp.int32)
    dst = edge_index[1].astype(jnp.int32)
    ew = edge_attr.astype(jnp.float32)

    pad_i = jnp.zeros((CHUNK,), jnp.int32)
    pad_f = jnp.zeros((CHUNK,), jnp.float32)
    src_p = jnp.concatenate([src, pad_i])
    dst_p = jnp.concatenate([dst, pad_i])
    ew_p = jnp.concatenate([ew, pad_f])

    degp = _deg_kernel(dst_p, ew_p)
    dis2d, idg2d = _dis_kernel(degp.reshape(NC, NP))
    dis = dis2d.reshape(NP)
    idgcol = idg2d.reshape(NP)[:N].reshape(N, 1)

    edata = _norm_kernel(src_p, dst_p, ew_p, dis)

    h1 = _mm(x, W1)
    acc1 = _agg_kernel(h1, edata)
    h2 = _combine_mm(acc1, h1, idgcol, b1, W2)
    acc2 = _agg_kernel(h2, edata)

    wm2p = jnp.zeros((D, D), jnp.float32).at[:, :bm2.shape[0]].set(Wm2)
    bm2p = jnp.zeros((D,), jnp.float32).at[:bm2.shape[0]].set(bm2)
    out = _head(acc2, h2, idgcol, b2, Wm1, bm1, wm2p, bm2p)
    return out[:, :bm2.shape[0]]


# speedup vs baseline: 1.0121x; 1.0121x over previous
import functools

import jax
import jax.numpy as jnp
from jax import lax
from jax.experimental import pallas as pl
from jax.experimental.pallas import tpu as pltpu
from jax.experimental.pallas import tpu_sc as plsc

N = 10000
E = 320000
D = 128
NP = 10240
NC = 2
NS = 16
NW = NC * NS
L = 16

CHUNK = 128
NCH = E // CHUNK
CPW = NCH // NW
REM = NCH - CPW * NW
CMAX = CPW + 1
EPW_MAX = CMAX * CHUNK
E_PAD = (NCH + 1) * CHUNK
SLICE = NP // NS

_mesh = plsc.VectorSubcoreMesh(
    core_axis_name="c", subcore_axis_name="s", num_cores=NC, num_subcores=NS)


def _wid_base_count(c, s):
    wid = s * NC + c
    cbase = wid * CPW + jnp.minimum(wid, REM)
    ccount = CPW + (wid < REM).astype(jnp.int32)
    return wid, cbase, ccount


@functools.partial(
    pl.kernel,
    out_type=jax.ShapeDtypeStruct((NC, 1, NP), jnp.float32),
    mesh=_mesh,
    compiler_params=pltpu.CompilerParams(needs_layout_passes=False),
    scratch_types=[
        pltpu.VMEM((EPW_MAX,), jnp.int32),
        pltpu.VMEM((EPW_MAX,), jnp.float32),
        pltpu.VMEM((NP,), jnp.float32),
        pltpu.VMEM((NS, SLICE), jnp.float32),
        pltpu.VMEM((SLICE,), jnp.float32),
        pltpu.VMEM_SHARED((NS, 1, NP), jnp.float32),
    ],
)
def _deg_kernel(dst_hbm, ew_hbm, degp_hbm, dstl, ewl, hist, redbuf, outbuf,
                shared):
    c = lax.axis_index("c")
    s = lax.axis_index("s")
    _, cbase, ccount = _wid_base_count(c, s)

    zero16 = jnp.zeros((L,), jnp.float32)

    def zero_body(i, _):
        hist[pl.ds(i * L, L)] = zero16
        return 0
    lax.fori_loop(0, NP // L, zero_body, 0)

    ebase = cbase * CHUNK
    pltpu.sync_copy(dst_hbm.at[pl.ds(ebase, EPW_MAX)], dstl)
    pltpu.sync_copy(ew_hbm.at[pl.ds(ebase, EPW_MAX)], ewl)

    def hist_body(g, _):
        dv = dstl[pl.ds(g * L, L)]
        wv = ewl[pl.ds(g * L, L)]
        plsc.addupdate_scatter(hist, [dv], wv)
        return 0
    lax.fori_loop(0, ccount * (CHUNK // L), hist_body, 0)

    pltpu.sync_copy(hist, shared.at[s, 0])
    plsc.subcore_barrier()

    pltpu.sync_copy(shared.at[:, 0, pl.ds(s * SLICE, SLICE)], redbuf)

    for i in range(SLICE // L):
        sl = pl.ds(i * L, L)
        acc = redbuf[0, sl]
        for r in range(1, NS):
            acc = acc + redbuf[r, sl]
        outbuf[sl] = acc
    pltpu.sync_copy(outbuf, degp_hbm.at[c, 0, pl.ds(s * SLICE, SLICE)])


PACKW = 3 * CHUNK


@functools.partial(
    pl.kernel,
    out_type=jax.ShapeDtypeStruct((E_PAD // CHUNK, 1, PACKW), jnp.int32),
    mesh=_mesh,
    compiler_params=pltpu.CompilerParams(needs_layout_passes=False),
    scratch_types=[
        pltpu.VMEM((EPW_MAX,), jnp.int32),
        pltpu.VMEM((EPW_MAX,), jnp.int32),
        pltpu.VMEM((EPW_MAX,), jnp.float32),
        pltpu.VMEM((CMAX, 1, PACKW), jnp.int32),
        pltpu.VMEM((NP,), jnp.float32),
    ],
)
def _norm_kernel(src_hbm, dst_hbm, ew_hbm, dis_hbm, edata_hbm, srcl, dstl,
                 ewl, epack, disv):
    c = lax.axis_index("c")
    s = lax.axis_index("s")
    wid, cbase, ccount = _wid_base_count(c, s)

    pltpu.sync_copy(dis_hbm, disv)
    ebase = cbase * CHUNK
    pltpu.sync_copy(src_hbm.at[pl.ds(ebase, EPW_MAX)], srcl)
    pltpu.sync_copy(dst_hbm.at[pl.ds(ebase, EPW_MAX)], dstl)
    pltpu.sync_copy(ew_hbm.at[pl.ds(ebase, EPW_MAX)], ewl)

    def body(j, _):
        for g in range(CHUNK // L):
            sl = pl.ds(g * L, L)
            s16 = srcl[pl.ds(j * CHUNK + g * L, L)]
            d16 = dstl[pl.ds(j * CHUNK + g * L, L)]
            w16 = ewl[pl.ds(j * CHUNK + g * L, L)]
            nrm = plsc.load_gather(disv, [s16]) * w16 * \
                plsc.load_gather(disv, [d16])
            epack[j, 0, sl] = s16
            epack[j, 0, pl.ds(CHUNK + g * L, L)] = d16
            epack[j, 0, pl.ds(2 * CHUNK + g * L, L)] = plsc.bitcast(
                nrm, jnp.int32)
        return 0
    lax.fori_loop(0, ccount, body, 0)

    pltpu.sync_copy(epack.at[pl.ds(0, CPW)], edata_hbm.at[pl.ds(cbase, CPW)])

    @pl.when(wid < REM)
    def _():
        pltpu.sync_copy(epack.at[pl.ds(CPW, 1)],
                        edata_hbm.at[pl.ds(cbase + CPW, 1)])


ROWS_A = 632
ROWS_LAST = N - 15 * ROWS_A


def _acc_slices(s):
    base = s * ROWS_A
    return base


@functools.partial(
    pl.kernel,
    out_type=jax.ShapeDtypeStruct((NC, N, D), jnp.float32),
    mesh=_mesh,
    compiler_params=pltpu.CompilerParams(needs_layout_passes=False),
    scratch_types=[
        pltpu.VMEM((1, PACKW), jnp.int32),
        pltpu.VMEM((1, PACKW), jnp.int32),
        pltpu.VMEM((1, PACKW), jnp.int32),
        pltpu.VMEM((1, PACKW), jnp.int32),
        pltpu.VMEM((CHUNK, D), jnp.float32),
        pltpu.VMEM((CHUNK, D), jnp.float32),
        pltpu.VMEM((CHUNK, D), jnp.float32),
        pltpu.VMEM_SHARED((N, D), jnp.float32),
        pltpu.SemaphoreType.DMA,
        pltpu.SemaphoreType.DMA,
        pltpu.SemaphoreType.DMA,
        pltpu.SemaphoreType.DMA,
        pltpu.SemaphoreType.DMA,
        pltpu.SemaphoreType.DMA,
        pltpu.SemaphoreType.DMA,
        pltpu.SemaphoreType.DMA,
        pltpu.SemaphoreType.DMA,
        pltpu.SemaphoreType.DMA,
    ],
)
def _agg_kernel(h_hbm, edata_hbm, accp_hbm,
                ebA, ebB, ebC, ebD, msgA, msgB, msgC, acc_sh,
                csA, csB, csC, csD, gsA, gsB, gsC, ssA, ssB, ssC):
    c = lax.axis_index("c")
    s = lax.axis_index("s")
    _, cbase, ccount = _wid_base_count(c, s)

    zero16 = jnp.zeros((L,), jnp.float32)

    def zbody(i, _):
        for q in range(D // L):
            msgA[i, pl.ds(q * L, L)] = zero16
        return 0
    lax.fori_loop(0, CHUNK, zbody, 0)
    base = s * ROWS_A
    for k in range(4):
        pltpu.sync_copy(msgA, acc_sh.at[pl.ds(base + k * CHUNK, CHUNK)])

    @pl.when(s < 15)
    def _():
        pltpu.sync_copy(msgA.at[pl.ds(0, ROWS_A - 4 * CHUNK)],
                        acc_sh.at[pl.ds(base + 4 * CHUNK,
                                        ROWS_A - 4 * CHUNK)])

    @pl.when(s == 15)
    def _():
        pltpu.sync_copy(msgA.at[pl.ds(0, ROWS_LAST - 4 * CHUNK)],
                        acc_sh.at[pl.ds(base + 4 * CHUNK,
                                        ROWS_LAST - 4 * CHUNK)])
    plsc.subcore_barrier()

    pltpu.async_copy(edata_hbm.at[cbase], ebA, csA)
    pltpu.async_copy(edata_hbm.at[cbase + 1], ebB, csB)
    pltpu.make_async_copy(edata_hbm.at[cbase], ebA, csA).wait()
    pltpu.async_copy(h_hbm.at[ebA.at[0, pl.ds(0, CHUNK)]], msgA, gsA)

    ebufs = ((ebA, csA), (ebB, csB), (ebC, csC), (ebD, csD))
    mbufs = ((msgA, gsA, ssA), (msgB, gsB, ssB), (msgC, gsC, ssC))
    zz = jnp.zeros((L,), jnp.int32)

    def slot_body(ii, _):
        for t in range(12):
            i = ii * 12 + t
            eb_i, cs_i = ebufs[t % 4]
            msg_i, gs_i, ss_i = mbufs[t % 3]
            eb_n, cs_n = ebufs[(t + 1) % 4]
            msg_n, gs_n, ss_n = mbufs[(t + 1) % 3]
            eb_c, cs_c = ebufs[(t + 2) % 4]

            @pl.when(i < ccount)
            def _():
                @pl.when(i + 1 < ccount)
                def _():
                    pltpu.make_async_copy(
                        edata_hbm.at[cbase + i + 1], eb_n, cs_n).wait()

                    @pl.when(i >= 2)
                    def _():
                        pltpu.make_async_copy(
                            msg_n,
                            acc_sh.at[eb_c.at[0, pl.ds(CHUNK, CHUNK)]],
                            ss_n).wait()
                    pltpu.async_copy(
                        h_hbm.at[eb_n.at[0, pl.ds(0, CHUNK)]], msg_n, gs_n)

                @pl.when(i + 2 < ccount)
                def _():
                    pltpu.async_copy(
                        edata_hbm.at[cbase + i + 2], eb_c, cs_c)

                pltpu.make_async_copy(
                    h_hbm.at[eb_i.at[0, pl.ds(0, CHUNK)]], msg_i,
                    gs_i).wait()

                @plsc.parallel_loop(0, CHUNK, unroll=4)
                def _(k):
                    ns = plsc.bitcast(
                        plsc.load_gather(
                            eb_i, [zz, jnp.full((L,), 2 * CHUNK + k,
                                                jnp.int32)]),
                        jnp.float32)
                    for q in range(D // L):
                        sl = pl.ds(q * L, L)
                        msg_i[k, sl] = msg_i[k, sl] * ns

                pltpu.async_copy(msg_i,
                                 acc_sh.at[eb_i.at[0, pl.ds(CHUNK, CHUNK)]],
                                 ss_i, add=True)
        return 0
    lax.fori_loop(0, (ccount + 11) // 12, slot_body, 0)

    for t in range(3):
        pltpu.make_async_copy(
            mbufs[t][0], acc_sh.at[ebA.at[0, pl.ds(CHUNK, CHUNK)]],
            mbufs[t][2]).wait()

    plsc.subcore_barrier()

    @pl.when(s < 15)
    def _():
        pltpu.sync_copy(acc_sh.at[pl.ds(base, ROWS_A)],
                        accp_hbm.at[c, pl.ds(base, ROWS_A)])

    @pl.when(s == 15)
    def _():
        pltpu.sync_copy(acc_sh.at[pl.ds(base, ROWS_LAST)],
                        accp_hbm.at[c, pl.ds(base, ROWS_LAST)])


def _dis_body(degp_ref, dis_ref, idg_ref):
    deg = degp_ref[0] + degp_ref[1] + 1.0
    dis_ref[...] = lax.rsqrt(deg)
    idg_ref[...] = 1.0 / deg


def _dis_kernel(degp):
    return pl.pallas_call(
        _dis_body,
        out_shape=[jax.ShapeDtypeStruct((NP // 128, 128), jnp.float32),
                   jax.ShapeDtypeStruct((NP // 128, 128), jnp.float32)],
    )(degp.reshape(NC, NP // 128, 128))


_BLK = 2000


def _mm_body(x_ref, w_ref, o_ref):
    o_ref[...] = jnp.dot(x_ref[...], w_ref[...],
                         preferred_element_type=jnp.float32)


def _mm(x, w):
    n = x.shape[0]
    return pl.pallas_call(
        _mm_body,
        grid=(n // _BLK,),
        in_specs=[pl.BlockSpec((_BLK, D), lambda i: (i, 0)),
                  pl.BlockSpec((D, D), lambda i: (0, 0))],
        out_specs=pl.BlockSpec((_BLK, D), lambda i: (i, 0)),
        out_shape=jax.ShapeDtypeStruct((n, D), jnp.float32),
    )(x, w)


def _comb_body(accp_ref, h_ref, idg_ref, b_ref, w_ref, o_ref):
    z = (accp_ref[0] + accp_ref[1] + idg_ref[...] * h_ref[...] + b_ref[...])
    z = jnp.maximum(z, 0.0)
    o_ref[...] = jnp.dot(z, w_ref[...], preferred_element_type=jnp.float32)


def _combine_mm(accp, h, idgcol, b, w):
    return pl.pallas_call(
        _comb_body,
        grid=(N // _BLK,),
        in_specs=[pl.BlockSpec((NC, _BLK, D), lambda i: (0, i, 0)),
                  pl.BlockSpec((_BLK, D), lambda i: (i, 0)),
                  pl.BlockSpec((_BLK, 1), lambda i: (i, 0)),
                  pl.BlockSpec((1, D), lambda i: (0, 0)),
                  pl.BlockSpec((D, D), lambda i: (0, 0))],
        out_specs=pl.BlockSpec((_BLK, D), lambda i: (i, 0)),
        out_shape=jax.ShapeDtypeStruct((N, D), jnp.float32),
    )(accp, h, idgcol, b.reshape(1, D), w)


def _head_body(accp_ref, h_ref, idg_ref, b_ref, wm1_ref, bm1_ref, wm2_ref,
               bm2_ref, o_ref):
    z = (accp_ref[0] + accp_ref[1] + idg_ref[...] * h_ref[...] + b_ref[...])
    z = jnp.maximum(z, 0.0)
    m = jnp.dot(z, wm1_ref[...], preferred_element_type=jnp.float32)
    m = jnp.maximum(m + bm1_ref[...], 0.0)
    o = jnp.dot(m, wm2_ref[...], preferred_element_type=jnp.float32)
    o_ref[...] = jax.nn.sigmoid(o + bm2_ref[...])


def _head(accp, h, idgcol, b2, wm1, bm1, wm2p, bm2p):
    return pl.pallas_call(
        _head_body,
        grid=(N // _BLK,),
        in_specs=[pl.BlockSpec((NC, _BLK, D), lambda i: (0, i, 0)),
                  pl.BlockSpec((_BLK, D), lambda i: (i, 0)),
                  pl.BlockSpec((_BLK, 1), lambda i: (i, 0)),
                  pl.BlockSpec((1, D), lambda i: (0, 0)),
                  pl.BlockSpec((D, D), lambda i: (0, 0)),
                  pl.BlockSpec((1, D), lambda i: (0, 0)),
                  pl.BlockSpec((D, D), lambda i: (0, 0)),
                  pl.BlockSpec((1, D), lambda i: (0, 0))],
        out_specs=pl.BlockSpec((_BLK, D), lambda i: (i, 0)),
        out_shape=jax.ShapeDtypeStruct((N, D), jnp.float32),
    )(accp, h, idgcol, b2.reshape(1, D), wm1, bm1.reshape(1, D), wm2p,
      bm2p.reshape(1, D))


def kernel(x, edge_index, edge_attr, W1, b1, W2, b2, Wm1, bm1, Wm2, bm2):
    src = edge_index[0].astype(jnp.int32)
    dst = edge_index[1].astype(jnp.int32)
    ew = edge_attr.astype(jnp.float32)

    pad_i = jnp.zeros((CHUNK,), jnp.int32)
    pad_f = jnp.zeros((CHUNK,), jnp.float32)
    src_p = jnp.concatenate([src, pad_i])
    dst_p = jnp.concatenate([dst, pad_i])
    ew_p = jnp.concatenate([ew, pad_f])

    degp = _deg_kernel(dst_p, ew_p)
    dis2d, idg2d = _dis_kernel(degp.reshape(NC, NP))
    dis = dis2d.reshape(NP)
    idgcol = idg2d.reshape(NP)[:N].reshape(N, 1)

    edata = _norm_kernel(src_p, dst_p, ew_p, dis)

    h1 = _mm(x, W1)
    acc1 = _agg_kernel(h1, edata)
    h2 = _combine_mm(acc1, h1, idgcol, b1, W2)
    acc2 = _agg_kernel(h2, edata)

    wm2p = jnp.zeros((D, D), jnp.float32).at[:, :bm2.shape[0]].set(Wm2)
    bm2p = jnp.zeros((D,), jnp.float32).at[:bm2.shape[0]].set(bm2)
    out = _head(acc2, h2, idgcol, b2, Wm1, bm1, wm2p, bm2p)
    return out[:, :bm2.shape[0]]


# speedup vs baseline: 1.0160x; 1.0038x over previous
import functools

import jax
import jax.numpy as jnp
from jax import lax
from jax.experimental import pallas as pl
from jax.experimental.pallas import tpu as pltpu
from jax.experimental.pallas import tpu_sc as plsc

N = 10000
E = 320000
D = 128
NP = 10240
NC = 2
NS = 16
NW = NC * NS
L = 16

CHUNK = 128
NCH = E // CHUNK
CPW = NCH // NW
REM = NCH - CPW * NW
CMAX = CPW + 1
EPW_MAX = CMAX * CHUNK
E_PAD = (NCH + 1) * CHUNK
SLICE = NP // NS

_mesh = plsc.VectorSubcoreMesh(
    core_axis_name="c", subcore_axis_name="s", num_cores=NC, num_subcores=NS)


def _wid_base_count(c, s):
    wid = s * NC + c
    cbase = wid * CPW + jnp.minimum(wid, REM)
    ccount = CPW + (wid < REM).astype(jnp.int32)
    return wid, cbase, ccount


PACKW = 3 * CHUNK
EPT_H = E // NS
HALF_H = EPT_H // 2


def _rsqrt16(x):
    i = plsc.bitcast(x, jnp.int32)
    i = jnp.int32(0x5F3759DF) - lax.shift_right_arithmetic(i, 1)
    y = plsc.bitcast(i, jnp.float32)
    for _ in range(3):
        y = y * (1.5 - 0.5 * x * y * y)
    return y


@functools.partial(
    pl.kernel,
    out_type=[jax.ShapeDtypeStruct((E_PAD // CHUNK, 1, PACKW), jnp.int32),
              jax.ShapeDtypeStruct((NP,), jnp.float32)],
    mesh=_mesh,
    compiler_params=pltpu.CompilerParams(needs_layout_passes=False),
    scratch_types=[
        pltpu.VMEM((EPW_MAX,), jnp.int32),
        pltpu.VMEM((EPW_MAX,), jnp.int32),
        pltpu.VMEM((EPW_MAX,), jnp.float32),
        pltpu.VMEM((CMAX, 1, PACKW), jnp.int32),
        pltpu.VMEM((NP,), jnp.float32),
        pltpu.VMEM((NS, SLICE), jnp.float32),
        pltpu.VMEM((SLICE,), jnp.float32),
        pltpu.VMEM_SHARED((NS, 1, NP), jnp.float32),
        pltpu.VMEM_SHARED((NP,), jnp.float32),
    ],
)
def _prep_kernel(src_hbm, dst_hbm, ew_hbm, edata_hbm, idg_hbm, srcl, dstl,
                 ewl, epack, disv, redbuf, outbuf, sh_hist, sh_dis):
    c = lax.axis_index("c")
    s = lax.axis_index("s")
    wid, cbase, ccount = _wid_base_count(c, s)

    zero16 = jnp.zeros((L,), jnp.float32)

    def zero_body(i, _):
        disv[pl.ds(i * L, L)] = zero16
        return 0
    lax.fori_loop(0, NP // L, zero_body, 0)

    for p in range(2):
        hbase = s * EPT_H + p * HALF_H
        pltpu.sync_copy(dst_hbm.at[pl.ds(hbase, HALF_H)],
                        dstl.at[pl.ds(0, HALF_H)])
        pltpu.sync_copy(ew_hbm.at[pl.ds(hbase, HALF_H)],
                        ewl.at[pl.ds(0, HALF_H)])

        def hist_body(g, _):
            dv = dstl[pl.ds(g * L, L)]
            wv = ewl[pl.ds(g * L, L)]
            plsc.addupdate_scatter(disv, [dv], wv)
            return 0
        lax.fori_loop(0, HALF_H // L, hist_body, 0)

    pltpu.sync_copy(disv, sh_hist.at[s, 0])
    plsc.subcore_barrier()

    pltpu.sync_copy(sh_hist.at[:, 0, pl.ds(s * SLICE, SLICE)], redbuf)
    for i in range(SLICE // L):
        sl = pl.ds(i * L, L)
        acc = redbuf[0, sl]
        for r in range(1, NS):
            acc = acc + redbuf[r, sl]
        outbuf[sl] = _rsqrt16(acc + 1.0)
    pltpu.sync_copy(outbuf, sh_dis.at[pl.ds(s * SLICE, SLICE)])

    for i in range(SLICE // L):
        sl = pl.ds(i * L, L)
        d16 = outbuf[sl]
        outbuf[sl] = d16 * d16

    @pl.when(c == 0)
    def _():
        pltpu.sync_copy(outbuf, idg_hbm.at[pl.ds(s * SLICE, SLICE)])
    plsc.subcore_barrier()
    pltpu.sync_copy(sh_dis, disv)

    ebase = cbase * CHUNK
    pltpu.sync_copy(src_hbm.at[pl.ds(ebase, EPW_MAX)], srcl)
    pltpu.sync_copy(dst_hbm.at[pl.ds(ebase, EPW_MAX)], dstl)
    pltpu.sync_copy(ew_hbm.at[pl.ds(ebase, EPW_MAX)], ewl)

    def body(j, _):
        for g in range(CHUNK // L):
            sl = pl.ds(g * L, L)
            s16 = srcl[pl.ds(j * CHUNK + g * L, L)]
            d16 = dstl[pl.ds(j * CHUNK + g * L, L)]
            w16 = ewl[pl.ds(j * CHUNK + g * L, L)]
            nrm = plsc.load_gather(disv, [s16]) * w16 * \
                plsc.load_gather(disv, [d16])
            epack[j, 0, sl] = s16
            epack[j, 0, pl.ds(CHUNK + g * L, L)] = d16
            epack[j, 0, pl.ds(2 * CHUNK + g * L, L)] = plsc.bitcast(
                nrm, jnp.int32)
        return 0
    lax.fori_loop(0, ccount, body, 0)

    pltpu.sync_copy(epack.at[pl.ds(0, CPW)], edata_hbm.at[pl.ds(cbase, CPW)])

    @pl.when(wid < REM)
    def _():
        pltpu.sync_copy(epack.at[pl.ds(CPW, 1)],
                        edata_hbm.at[pl.ds(cbase + CPW, 1)])


ROWS_A = 632
ROWS_LAST = N - 15 * ROWS_A


def _acc_slices(s):
    base = s * ROWS_A
    return base


@functools.partial(
    pl.kernel,
    out_type=jax.ShapeDtypeStruct((NC, N, D), jnp.float32),
    mesh=_mesh,
    compiler_params=pltpu.CompilerParams(needs_layout_passes=False),
    scratch_types=[
        pltpu.VMEM((1, PACKW), jnp.int32),
        pltpu.VMEM((1, PACKW), jnp.int32),
        pltpu.VMEM((1, PACKW), jnp.int32),
        pltpu.VMEM((1, PACKW), jnp.int32),
        pltpu.VMEM((CHUNK, D), jnp.float32),
        pltpu.VMEM((CHUNK, D), jnp.float32),
        pltpu.VMEM((CHUNK, D), jnp.float32),
        pltpu.VMEM_SHARED((N, D), jnp.float32),
        pltpu.SemaphoreType.DMA,
        pltpu.SemaphoreType.DMA,
        pltpu.SemaphoreType.DMA,
        pltpu.SemaphoreType.DMA,
        pltpu.SemaphoreType.DMA,
        pltpu.SemaphoreType.DMA,
        pltpu.SemaphoreType.DMA,
        pltpu.SemaphoreType.DMA,
        pltpu.SemaphoreType.DMA,
        pltpu.SemaphoreType.DMA,
    ],
)
def _agg_kernel(h_hbm, edata_hbm, accp_hbm,
                ebA, ebB, ebC, ebD, msgA, msgB, msgC, acc_sh,
                csA, csB, csC, csD, gsA, gsB, gsC, ssA, ssB, ssC):
    c = lax.axis_index("c")
    s = lax.axis_index("s")
    _, cbase, ccount = _wid_base_count(c, s)

    zero16 = jnp.zeros((L,), jnp.float32)

    def zbody(i, _):
        for q in range(D // L):
            msgA[i, pl.ds(q * L, L)] = zero16
        return 0
    lax.fori_loop(0, CHUNK, zbody, 0)
    base = s * ROWS_A
    for k in range(4):
        pltpu.sync_copy(msgA, acc_sh.at[pl.ds(base + k * CHUNK, CHUNK)])

    @pl.when(s < 15)
    def _():
        pltpu.sync_copy(msgA.at[pl.ds(0, ROWS_A - 4 * CHUNK)],
                        acc_sh.at[pl.ds(base + 4 * CHUNK,
                                        ROWS_A - 4 * CHUNK)])

    @pl.when(s == 15)
    def _():
        pltpu.sync_copy(msgA.at[pl.ds(0, ROWS_LAST - 4 * CHUNK)],
                        acc_sh.at[pl.ds(base + 4 * CHUNK,
                                        ROWS_LAST - 4 * CHUNK)])
    plsc.subcore_barrier()

    pltpu.async_copy(edata_hbm.at[cbase], ebA, csA)
    pltpu.async_copy(edata_hbm.at[cbase + 1], ebB, csB)
    pltpu.make_async_copy(edata_hbm.at[cbase], ebA, csA).wait()
    pltpu.async_copy(h_hbm.at[ebA.at[0, pl.ds(0, CHUNK)]], msgA, gsA)

    ebufs = ((ebA, csA), (ebB, csB), (ebC, csC), (ebD, csD))
    mbufs = ((msgA, gsA, ssA), (msgB, gsB, ssB), (msgC, gsC, ssC))
    zz = jnp.zeros((L,), jnp.int32)

    def slot_body(ii, _):
        for t in range(12):
            i = ii * 12 + t
            eb_i, cs_i = ebufs[t % 4]
            msg_i, gs_i, ss_i = mbufs[t % 3]
            eb_n, cs_n = ebufs[(t + 1) % 4]
            msg_n, gs_n, ss_n = mbufs[(t + 1) % 3]
            eb_c, cs_c = ebufs[(t + 2) % 4]

            @pl.when(i < ccount)
            def _():
                @pl.when(i + 1 < ccount)
                def _():
                    pltpu.make_async_copy(
                        edata_hbm.at[cbase + i + 1], eb_n, cs_n).wait()

                    @pl.when(i >= 2)
                    def _():
                        pltpu.make_async_copy(
                            msg_n,
                            acc_sh.at[eb_c.at[0, pl.ds(CHUNK, CHUNK)]],
                            ss_n).wait()
                    pltpu.async_copy(
                        h_hbm.at[eb_n.at[0, pl.ds(0, CHUNK)]], msg_n, gs_n)

                @pl.when(i + 2 < ccount)
                def _():
                    pltpu.async_copy(
                        edata_hbm.at[cbase + i + 2], eb_c, cs_c)

                pltpu.make_async_copy(
                    h_hbm.at[eb_i.at[0, pl.ds(0, CHUNK)]], msg_i,
                    gs_i).wait()

                @plsc.parallel_loop(0, CHUNK, unroll=4)
                def _(k):
                    ns = plsc.bitcast(
                        plsc.load_gather(
                            eb_i, [zz, jnp.full((L,), 2 * CHUNK + k,
                                                jnp.int32)]),
                        jnp.float32)
                    for q in range(D // L):
                        sl = pl.ds(q * L, L)
                        msg_i[k, sl] = msg_i[k, sl] * ns

                pltpu.async_copy(msg_i,
                                 acc_sh.at[eb_i.at[0, pl.ds(CHUNK, CHUNK)]],
                                 ss_i, add=True)
        return 0
    lax.fori_loop(0, (ccount + 11) // 12, slot_body, 0)

    for t in range(3):
        pltpu.make_async_copy(
            mbufs[t][0], acc_sh.at[ebA.at[0, pl.ds(CHUNK, CHUNK)]],
            mbufs[t][2]).wait()

    plsc.subcore_barrier()

    @pl.when(s < 15)
    def _():
        pltpu.sync_copy(acc_sh.at[pl.ds(base, ROWS_A)],
                        accp_hbm.at[c, pl.ds(base, ROWS_A)])

    @pl.when(s == 15)
    def _():
        pltpu.sync_copy(acc_sh.at[pl.ds(base, ROWS_LAST)],
                        accp_hbm.at[c, pl.ds(base, ROWS_LAST)])


_BLK = 2000


def _mm_body(x_ref, w_ref, o_ref):
    o_ref[...] = jnp.dot(x_ref[...], w_ref[...],
                         preferred_element_type=jnp.float32)


def _mm(x, w):
    n = x.shape[0]
    return pl.pallas_call(
        _mm_body,
        grid=(n // _BLK,),
        in_specs=[pl.BlockSpec((_BLK, D), lambda i: (i, 0)),
                  pl.BlockSpec((D, D), lambda i: (0, 0))],
        out_specs=pl.BlockSpec((_BLK, D), lambda i: (i, 0)),
        out_shape=jax.ShapeDtypeStruct((n, D), jnp.float32),
    )(x, w)


def _comb_body(accp_ref, h_ref, idg_ref, b_ref, w_ref, o_ref):
    z = (accp_ref[0] + accp_ref[1] + idg_ref[...] * h_ref[...] + b_ref[...])
    z = jnp.maximum(z, 0.0)
    o_ref[...] = jnp.dot(z, w_ref[...], preferred_element_type=jnp.float32)


def _combine_mm(accp, h, idgcol, b, w):
    return pl.pallas_call(
        _comb_body,
        grid=(N // _BLK,),
        in_specs=[pl.BlockSpec((NC, _BLK, D), lambda i: (0, i, 0)),
                  pl.BlockSpec((_BLK, D), lambda i: (i, 0)),
                  pl.BlockSpec((_BLK, 1), lambda i: (i, 0)),
                  pl.BlockSpec((1, D), lambda i: (0, 0)),
                  pl.BlockSpec((D, D), lambda i: (0, 0))],
        out_specs=pl.BlockSpec((_BLK, D), lambda i: (i, 0)),
        out_shape=jax.ShapeDtypeStruct((N, D), jnp.float32),
    )(accp, h, idgcol, b.reshape(1, D), w)


def _head_body(accp_ref, h_ref, idg_ref, b_ref, wm1_ref, bm1_ref, wm2_ref,
               bm2_ref, o_ref):
    z = (accp_ref[0] + accp_ref[1] + idg_ref[...] * h_ref[...] + b_ref[...])
    z = jnp.maximum(z, 0.0)
    m = jnp.dot(z, wm1_ref[...], preferred_element_type=jnp.float32)
    m = jnp.maximum(m + bm1_ref[...], 0.0)
    o = jnp.dot(m, wm2_ref[...], preferred_element_type=jnp.float32)
    o_ref[...] = jax.nn.sigmoid(o + bm2_ref[...])


def _head(accp, h, idgcol, b2, wm1, bm1, wm2p, bm2p):
    return pl.pallas_call(
        _head_body,
        grid=(N // _BLK,),
        in_specs=[pl.BlockSpec((NC, _BLK, D), lambda i: (0, i, 0)),
                  pl.BlockSpec((_BLK, D), lambda i: (i, 0)),
                  pl.BlockSpec((_BLK, 1), lambda i: (i, 0)),
                  pl.BlockSpec((1, D), lambda i: (0, 0)),
                  pl.BlockSpec((D, D), lambda i: (0, 0)),
                  pl.BlockSpec((1, D), lambda i: (0, 0)),
                  pl.BlockSpec((D, D), lambda i: (0, 0)),
                  pl.BlockSpec((1, D), lambda i: (0, 0))],
        out_specs=pl.BlockSpec((_BLK, D), lambda i: (i, 0)),
        out_shape=jax.ShapeDtypeStruct((N, D), jnp.float32),
    )(accp, h, idgcol, b2.reshape(1, D), wm1, bm1.reshape(1, D), wm2p,
      bm2p.reshape(1, D))


def kernel(x, edge_index, edge_attr, W1, b1, W2, b2, Wm1, bm1, Wm2, bm2):
    src = edge_index[0].astype(jnp.int32)
    dst = edge_index[1].astype(jnp.int32)
    ew = edge_attr.astype(jnp.float32)

    pad_i = jnp.zeros((CHUNK,), jnp.int32)
    pad_f = jnp.zeros((CHUNK,), jnp.float32)
    src_p = jnp.concatenate([src, pad_i])
    dst_p = jnp.concatenate([dst, pad_i])
    ew_p = jnp.concatenate([ew, pad_f])

    edata, idg = _prep_kernel(src_p, dst_p, ew_p)
    idgcol = idg[:N].reshape(N, 1)

    h1 = _mm(x, W1)
    acc1 = _agg_kernel(h1, edata)
    h2 = _combine_mm(acc1, h1, idgcol, b1, W2)
    acc2 = _agg_kernel(h2, edata)

    wm2p = jnp.zeros((D, D), jnp.float32).at[:, :bm2.shape[0]].set(Wm2)
    bm2p = jnp.zeros((D,), jnp.float32).at[:bm2.shape[0]].set(bm2)
    out = _head(acc2, h2, idgcol, b2, Wm1, bm1, wm2p, bm2p)
    return out[:, :bm2.shape[0]]
